# baseline (speedup 1.0000x reference)
import functools

import jax
import jax.numpy as jnp
from jax import lax
from jax.experimental import pallas as pl
from jax.experimental.pallas import tpu as pltpu
from jax.experimental.pallas import tpu_sc as plsc

B = 16384
F = 26
K = 16
EMBED_OUT = F * K
BF = B * F
NV = 1000012

NC = 2
NS = 16
NW = NC * NS
PER_W = BF // NW
CHUNK = 128
NCHUNK = PER_W // CHUNK

NPAIR = 7813
NPAIR_FULL = 7812
TAIL = NV - NPAIR_FULL * 128
NGROUP = NPAIR * 16
OUT_ROWS = BF * K // 128


_BASEC = 244
_EXTRA0 = _BASEC * NW
CP = 4
_NCH_PRE = _BASEC // CP


def _permute_tile(src_ref, dst_ref, src_col0, dst_row0):
    lane = lax.iota(jnp.int32, 16)
    rbase = jax.lax.shift_right_logical(lane, 3) + dst_row0
    cbase = jax.lax.bitwise_and(lane, 7) * 16
    for q in range(8):
        r = rbase + 2 * q
        for k in range(16):
            v = src_ref[k, pl.ds(src_col0 + q * 16, 16)]
            plsc.store_scatter(dst_ref, [r, cbase + k], v)


def _prepass_body(embt_hbm, tailp_hbm, tab_hbm, tile_v, piece_v,
                  sem_in, sem_out):
    wid = lax.axis_index("s") * NC + lax.axis_index("c")
    pstart = _BASEC * wid

    def in_cp(c, b):
        p0 = pstart + c * CP
        return pltpu.make_async_copy(
            embt_hbm.at[:, pl.ds(pl.multiple_of(p0 * 128, 128), CP * 128)], tile_v.at[b], sem_in[b])

    def out_cp(c, b):
        p0 = pstart + c * CP
        return pltpu.make_async_copy(
            piece_v.at[b],
            tab_hbm.at[pl.ds(pl.multiple_of(p0 * 16, 16), CP * 16)],
            sem_out[b])

    in_cp(0, 0).start()
    in_cp(1, 1).start()

    @pl.loop(0, _NCH_PRE // 3 + 1)
    def _j(j):
        for b in range(3):
            c = 3 * j + b

            @pl.when(c < _NCH_PRE)
            def _do():
                in_cp(c, b).wait()

                @pl.when(c + 2 < _NCH_PRE)
                def _pf():
                    in_cp(c + 2, (b + 2) % 3).start()

                @pl.when(c > 2)
                def _wo():
                    out_cp(c - 3, b).wait()

                for pp in range(CP):
                    _permute_tile(tile_v.at[b], piece_v.at[b], pp * 128, pp * 16)
                out_cp(c, b).start()

    out_cp(_NCH_PRE - 3, (_NCH_PRE - 3) % 3).wait()
    out_cp(_NCH_PRE - 2, (_NCH_PRE - 2) % 3).wait()
    out_cp(_NCH_PRE - 1, (_NCH_PRE - 1) % 3).wait()

    nrem = _BASEC - _NCH_PRE * CP
    if nrem:
        p0 = pstart + _NCH_PRE * CP
        pltpu.async_copy(
            embt_hbm.at[:, pl.ds(pl.multiple_of(p0 * 128, 128), nrem * 128)],
            tile_v.at[0, :, pl.ds(0, nrem * 128)], sem_in[0]).wait()
        for pp in range(nrem):
            _permute_tile(tile_v.at[0], piece_v.at[0], pp * 128, pp * 16)
        pltpu.async_copy(
            piece_v.at[0, pl.ds(0, nrem * 16)],
            tab_hbm.at[pl.ds(pl.multiple_of(p0 * 16, 16), nrem * 16)],
            sem_out[0]).wait()

    @pl.when(wid < 4)
    def _extra():
        p = _EXTRA0 + wid
        pltpu.async_copy(
            embt_hbm.at[:, pl.ds(pl.multiple_of(p * 128, 128), 128)],
            tile_v.at[0, :, pl.ds(0, 128)], sem_in[0]).wait()
        _permute_tile(tile_v.at[0], piece_v.at[0], 0, 0)
        pltpu.async_copy(
            piece_v.at[0, pl.ds(0, 16)],
            tab_hbm.at[pl.ds(pl.multiple_of(p * 16, 16), 16)],
            sem_out[0]).wait()

    @pl.when(wid == 4)
    def _tail():
        pltpu.async_copy(tailp_hbm, piece_v.at[0, pl.ds(0, 16)],
                         sem_in[0]).wait()
        pltpu.async_copy(
            piece_v.at[0, pl.ds(0, 16)],
            tab_hbm.at[pl.ds(NPAIR_FULL * 16, 16)], sem_out[0]).wait()


@functools.cache
def _build_prepass():
    mesh = plsc.VectorSubcoreMesh(
        core_axis_name="c", subcore_axis_name="s",
        num_cores=NC, num_subcores=NS,
    )
    return pl.kernel(
        _prepass_body,
        out_type=jax.ShapeDtypeStruct((NGROUP, 128), jnp.float32),
        mesh=mesh,
        compiler_params=pltpu.CompilerParams(use_tc_tiling_on_sc=True, needs_layout_passes=False),
        scratch_types=(
            pltpu.VMEM((3, 16, CP * 128), jnp.float32),
            pltpu.VMEM((3, CP * 16, 128), jnp.float32),
            (pltpu.SemaphoreType.DMA, pltpu.SemaphoreType.DMA,
             pltpu.SemaphoreType.DMA),
            (pltpu.SemaphoreType.DMA, pltpu.SemaphoreType.DMA,
             pltpu.SemaphoreType.DMA),
        ),
    )


SLICES = ((0, 78), (78 * CHUNK * NW, 26))


def _gather_body(goff, nch, idx_hbm, tab_hbm, fct_hbm, rows_hbm, fcv_hbm,
                 idx_v, g_v, grp_v, out_v, fcv_v,
                 sem_g, sem_f, sem_o, sem_fo):
    wid = lax.axis_index("s") * NC + lax.axis_index("c")
    per_w = nch * CHUNK
    base = wid * per_w
    gbase = goff + base
    pltpu.sync_copy(idx_hbm.at[pl.ds(gbase, per_w)], idx_v)

    def build_g(c, b):
        off = c * CHUNK
        for q in range(CHUNK // 16):
            iv = idx_v[pl.ds(off + q * 16, 16)]
            g_v[b, pl.ds(q * 16, 16)] = jax.lax.shift_right_logical(iv, 3)

    def g_cp(c, b):
        return pltpu.make_async_copy(tab_hbm.at[g_v.at[b]], grp_v.at[b],
                                     sem_g[b])

    def f_cp(c, b):
        off = c * CHUNK
        return pltpu.make_async_copy(
            fct_hbm.at[idx_v.at[pl.ds(off, CHUNK)]], fcv_v.at[b], sem_f[b])

    def o_cp(c, b):
        off = c * CHUNK
        return pltpu.make_async_copy(
            out_v.at[b],
            rows_hbm.at[pl.ds(pl.multiple_of((base + off) // 8, 16), 16)],
            sem_o[b])

    def fo_cp(c, b):
        off = c * CHUNK
        return pltpu.make_async_copy(
            fcv_v.at[b], fcv_hbm.at[pl.ds(base + off, CHUNK)], sem_fo[b])

    build_g(0, 0)
    g_cp(0, 0).start()
    f_cp(0, 0).start()

    @pl.loop(0, nch // 2)
    def _j(j):
        for b in range(2):
            c = 2 * j + b
            @pl.when(c + 1 < nch)
            def _pf():
                @pl.when(c > 0)
                def _wfo():
                    fo_cp(c - 1, 1 - b).wait()
                build_g(c + 1, 1 - b)
                g_cp(c + 1, 1 - b).start()
                f_cp(c + 1, 1 - b).start()

            g_cp(c, b).wait()
            f_cp(c, b).wait()

            @pl.when(c > 1)
            def _wo():
                o_cp(c - 2, b).wait()

            off = c * CHUNK
            lane = lax.iota(jnp.int32, 16)
            rbase = jax.lax.shift_right_logical(lane, 3)
            cbase = jax.lax.bitwise_and(lane, 7) * 16
            for q in range(CHUNK // 16):
                iv = idx_v[pl.ds(off + q * 16, 16)]
                sub = jax.lax.bitwise_and(iv, 7) * 16
                j16 = jnp.full((16,), q * 16, jnp.int32) + lane
                r = rbase + 2 * q
                for k in range(16):
                    v = plsc.load_gather(grp_v.at[b], [j16, sub + k])
                    plsc.store_scatter(out_v.at[b], [r, cbase + k], v)
            o_cp(c, b).start()
            fo_cp(c, b).start()

    o_cp(nch - 2, 0).wait()
    o_cp(nch - 1, 1).wait()
    fo_cp(nch - 2, 0).wait()
    fo_cp(nch - 1, 1).wait()


@functools.cache
def _build_gather(goff, nch):
    mesh = plsc.VectorSubcoreMesh(
        core_axis_name="c", subcore_axis_name="s",
        num_cores=NC, num_subcores=NS,
    )
    bf_s = nch * CHUNK * NW
    return pl.kernel(
        functools.partial(_gather_body, goff, nch),
        out_type=(
            jax.ShapeDtypeStruct((bf_s * K // 128, 128), jnp.float32),
            jax.ShapeDtypeStruct((bf_s,), jnp.float32),
        ),
        mesh=mesh,
        compiler_params=pltpu.CompilerParams(use_tc_tiling_on_sc=True, needs_layout_passes=False),
        scratch_types=(
            pltpu.VMEM((nch * CHUNK,), jnp.int32),
            pltpu.VMEM((2, CHUNK), jnp.int32),
            pltpu.VMEM((2, CHUNK, 128), jnp.float32),
            pltpu.VMEM((2, 16, 128), jnp.float32),
            pltpu.VMEM((2, CHUNK), jnp.float32),
            (pltpu.SemaphoreType.DMA, pltpu.SemaphoreType.DMA),
            (pltpu.SemaphoreType.DMA, pltpu.SemaphoreType.DMA),
            (pltpu.SemaphoreType.DMA, pltpu.SemaphoreType.DMA),
            (pltpu.SemaphoreType.DMA, pltpu.SemaphoreType.DMA),
        ),
    )


BLK = 1024


def _mlp_body(e_ref, fc_ref, s_ref, w1_ref, b1_ref, w2_ref, b2_ref,
              w3_ref, b3_ref, wlin_ref, blin_ref, out_ref):
    e = e_ref[...]
    s = s_ref[...]
    sum_f = lax.dot_general(e, s, (((1,), (0,)), ((), ())),
                            preferred_element_type=jnp.float32)
    ssq = lax.dot_general(e * e, s, (((1,), (0,)), ((), ())),
                          preferred_element_type=jnp.float32)
    fm = 0.5 * jnp.sum(sum_f * sum_f - ssq, axis=1, keepdims=True)

    lin = jnp.sum(fc_ref[...], axis=1, keepdims=True)
    lin = lin * wlin_ref[0, 0] + blin_ref[0, 0]

    h = lax.dot_general(e, w1_ref[...], (((1,), (0,)), ((), ())),
                        preferred_element_type=jnp.float32)
    h = jnp.maximum(h + b1_ref[...], 0.0)
    h = lax.dot_general(h, w2_ref[...], (((1,), (0,)), ((), ())),
                        preferred_element_type=jnp.float32)
    h = jnp.maximum(h + b2_ref[...], 0.0)
    mlp = lax.dot_general(h, w3_ref[...], (((1,), (0,)), ((), ())),
                          preferred_element_type=jnp.float32)
    mlp = mlp + b3_ref[0, 0]

    z = lin + fm + mlp
    out_ref[...] = 1.0 / (1.0 + jnp.exp(-z))


def _tc_mlp(e, fcm, s, w1, b1, w2, b2, w3, b3, wlin, blin):
    nb = e.shape[0]
    grid = (nb // BLK,)
    fixed = lambda i: (0, 0)
    return pl.pallas_call(
        _mlp_body,
        grid=grid,
        in_specs=[
            pl.BlockSpec((BLK, EMBED_OUT), lambda i: (i, 0)),
            pl.BlockSpec((BLK, F), lambda i: (i, 0)),
            pl.BlockSpec((EMBED_OUT, K), fixed),
            pl.BlockSpec((EMBED_OUT, 128), fixed),
            pl.BlockSpec((1, 128), fixed),
            pl.BlockSpec((128, 64), fixed),
            pl.BlockSpec((1, 64), fixed),
            pl.BlockSpec((64, 1), fixed),
            pl.BlockSpec((1, 1), fixed),
            pl.BlockSpec((1, 1), fixed),
            pl.BlockSpec((1, 1), fixed),
        ],
        out_specs=pl.BlockSpec((BLK, 1), lambda i: (i, 0)),
        out_shape=jax.ShapeDtypeStruct((nb, 1), jnp.float32),
    )(e, fcm, s, w1, b1, w2, b2, w3, b3, wlin, blin)


def kernel(x, embedding, fc_table, w_lin, b_lin, W1, b1, W2, b2, W3, b3):
    xf = x.reshape(-1).astype(jnp.int32)
    tail_piece = jnp.pad(embedding[NPAIR_FULL * 128:],
                         ((0, 128 - TAIL), (0, 0))).reshape(16, 128)
    tab = _build_prepass()(embedding.T, tail_piece)
    fct = fc_table.reshape(-1)
    s = (jnp.arange(EMBED_OUT)[:, None] % K == jnp.arange(K)[None, :]
         ).astype(jnp.float32)
    outs = []
    for goff, nch in SLICES:
        rows, fcv = _build_gather(goff, nch)(xf, tab, fct)
        nb = nch * CHUNK * NW // F
        e = rows.reshape(nb, EMBED_OUT)
        fcm = fcv.reshape(nb, F)
        outs.append(_tc_mlp(e, fcm, s, W1, b1.reshape(1, -1),
                            W2, b2.reshape(1, -1), W3, b3.reshape(1, 1),
                            w_lin, b_lin.reshape(1, 1)))
    return jnp.concatenate(outs, axis=0)


# speedup vs baseline: 1.0692x; 1.0692x over previous
import functools

import jax
import jax.numpy as jnp
from jax import lax
from jax.experimental import pallas as pl
from jax.experimental.pallas import tpu as pltpu
from jax.experimental.pallas import tpu_sc as plsc

B = 16384
F = 26
K = 16
EMBED_OUT = F * K
BF = B * F
NV = 1000012

NC = 2
NS = 16
NW = NC * NS
PER_W = BF // NW
CHUNK = 128
NCHUNK = PER_W // CHUNK

NPAIR = 7813
NPAIR_FULL = 7812
TAIL = NV - NPAIR_FULL * 128
NGROUP = NPAIR * 16
OUT_ROWS = BF * K // 128


_BASEC = 244
_EXTRA0 = _BASEC * NW
CP = 4
_NCH_PRE = _BASEC // CP


def _permute_tile(src_ref, dst_ref, src_col0, dst_row0):
    lane = lax.iota(jnp.int32, 16)
    rbase = jax.lax.shift_right_logical(lane, 3) + dst_row0
    cbase = jax.lax.bitwise_and(lane, 7) * 16
    for q in range(8):
        r = rbase + 2 * q
        for k in range(16):
            v = src_ref[k, pl.ds(src_col0 + q * 16, 16)]
            plsc.store_scatter(dst_ref, [r, cbase + k], v)


def _prepass_body(embt_hbm, tailp_hbm, tab_hbm, tile_v, piece_v,
                  sem_in, sem_out):
    wid = lax.axis_index("s") * NC + lax.axis_index("c")
    pstart = _BASEC * wid

    def in_cp(c, b):
        p0 = pstart + c * CP
        return pltpu.make_async_copy(
            embt_hbm.at[:, pl.ds(pl.multiple_of(p0 * 128, 128), CP * 128)], tile_v.at[b], sem_in[b])

    def out_cp(c, b):
        p0 = pstart + c * CP
        return pltpu.make_async_copy(
            piece_v.at[b],
            tab_hbm.at[pl.ds(pl.multiple_of(p0 * 16, 16), CP * 16)],
            sem_out[b])

    in_cp(0, 0).start()

    @pl.loop(0, _NCH_PRE // 2 + 1)
    def _j(j):
        for b in range(2):
            c = 2 * j + b

            @pl.when(c < _NCH_PRE)
            def _do():
                in_cp(c, b).wait()

                @pl.when(c + 1 < _NCH_PRE)
                def _pf():
                    in_cp(c + 1, 1 - b).start()

                @pl.when(c > 1)
                def _wo():
                    out_cp(c - 2, b).wait()

                for pp in range(CP):
                    _permute_tile(tile_v.at[b], piece_v.at[b], pp * 128, pp * 16)
                out_cp(c, b).start()

    out_cp(_NCH_PRE - 2, _NCH_PRE % 2).wait()
    out_cp(_NCH_PRE - 1, (_NCH_PRE - 1) % 2).wait()

    nrem = _BASEC - _NCH_PRE * CP
    if nrem:
        p0 = pstart + _NCH_PRE * CP
        pltpu.async_copy(
            embt_hbm.at[:, pl.ds(pl.multiple_of(p0 * 128, 128), nrem * 128)],
            tile_v.at[0, :, pl.ds(0, nrem * 128)], sem_in[0]).wait()
        for pp in range(nrem):
            _permute_tile(tile_v.at[0], piece_v.at[0], pp * 128, pp * 16)
        pltpu.async_copy(
            piece_v.at[0, pl.ds(0, nrem * 16)],
            tab_hbm.at[pl.ds(pl.multiple_of(p0 * 16, 16), nrem * 16)],
            sem_out[0]).wait()

    @pl.when(wid < 4)
    def _extra():
        p = _EXTRA0 + wid
        pltpu.async_copy(
            embt_hbm.at[:, pl.ds(pl.multiple_of(p * 128, 128), 128)],
            tile_v.at[0, :, pl.ds(0, 128)], sem_in[0]).wait()
        _permute_tile(tile_v.at[0], piece_v.at[0], 0, 0)
        pltpu.async_copy(
            piece_v.at[0, pl.ds(0, 16)],
            tab_hbm.at[pl.ds(pl.multiple_of(p * 16, 16), 16)],
            sem_out[0]).wait()

    @pl.when(wid == 4)
    def _tail():
        pltpu.async_copy(tailp_hbm, piece_v.at[0, pl.ds(0, 16)],
                         sem_in[0]).wait()
        pltpu.async_copy(
            piece_v.at[0, pl.ds(0, 16)],
            tab_hbm.at[pl.ds(NPAIR_FULL * 16, 16)], sem_out[0]).wait()


@functools.cache
def _build_prepass():
    mesh = plsc.VectorSubcoreMesh(
        core_axis_name="c", subcore_axis_name="s",
        num_cores=NC, num_subcores=NS,
    )
    return pl.kernel(
        _prepass_body,
        out_type=jax.ShapeDtypeStruct((NGROUP, 128), jnp.float32),
        mesh=mesh,
        compiler_params=pltpu.CompilerParams(use_tc_tiling_on_sc=True, needs_layout_passes=False),
        scratch_types=(
            pltpu.VMEM((2, 16, CP * 128), jnp.float32),
            pltpu.VMEM((2, CP * 16, 128), jnp.float32),
            (pltpu.SemaphoreType.DMA, pltpu.SemaphoreType.DMA),
            (pltpu.SemaphoreType.DMA, pltpu.SemaphoreType.DMA),
        ),
    )


SLICES = ((0, 52), (52 * CHUNK * NW, 52))


def _gather_body(goff, nch, idx_hbm, tab_hbm, fct_hbm, rows_hbm, fcv_hbm,
                 idx_v, g_v, grp_v, out_v, fcv_v,
                 sem_g, sem_f, sem_o, sem_fo):
    wid = lax.axis_index("s") * NC + lax.axis_index("c")
    per_w = nch * CHUNK
    base = wid * per_w
    gbase = goff + base
    pltpu.sync_copy(idx_hbm.at[pl.ds(gbase, per_w)], idx_v)

    def build_g(c, b):
        off = c * CHUNK
        for q in range(CHUNK // 16):
            iv = idx_v[pl.ds(off + q * 16, 16)]
            g_v[b, pl.ds(q * 16, 16)] = jax.lax.shift_right_logical(iv, 3)

    def g_cp(c, b):
        return pltpu.make_async_copy(tab_hbm.at[g_v.at[b]], grp_v.at[b],
                                     sem_g[b])

    def f_cp(c, b):
        off = c * CHUNK
        return pltpu.make_async_copy(
            fct_hbm.at[idx_v.at[pl.ds(off, CHUNK)]], fcv_v.at[b], sem_f[b])

    def o_cp(c, b):
        off = c * CHUNK
        return pltpu.make_async_copy(
            out_v.at[b],
            rows_hbm.at[pl.ds(pl.multiple_of((base + off) // 8, 16), 16)],
            sem_o[b])

    def fo_cp(c, b):
        off = c * CHUNK
        return pltpu.make_async_copy(
            fcv_v.at[b], fcv_hbm.at[pl.ds(base + off, CHUNK)], sem_fo[b])

    build_g(0, 0)
    g_cp(0, 0).start()
    f_cp(0, 0).start()

    @pl.loop(0, nch // 2)
    def _j(j):
        for b in range(2):
            c = 2 * j + b
            @pl.when(c + 1 < nch)
            def _pf():
                @pl.when(c > 0)
                def _wfo():
                    fo_cp(c - 1, 1 - b).wait()
                build_g(c + 1, 1 - b)
                g_cp(c + 1, 1 - b).start()
                f_cp(c + 1, 1 - b).start()

            g_cp(c, b).wait()
            f_cp(c, b).wait()

            @pl.when(c > 1)
            def _wo():
                o_cp(c - 2, b).wait()

            off = c * CHUNK
            lane = lax.iota(jnp.int32, 16)
            rbase = jax.lax.shift_right_logical(lane, 3)
            cbase = jax.lax.bitwise_and(lane, 7) * 16
            for q in range(CHUNK // 16):
                iv = idx_v[pl.ds(off + q * 16, 16)]
                sub = jax.lax.bitwise_and(iv, 7) * 16
                j16 = jnp.full((16,), q * 16, jnp.int32) + lane
                r = rbase + 2 * q
                for k in range(16):
                    v = plsc.load_gather(grp_v.at[b], [j16, sub + k])
                    plsc.store_scatter(out_v.at[b], [r, cbase + k], v)
            o_cp(c, b).start()
            fo_cp(c, b).start()

    o_cp(nch - 2, 0).wait()
    o_cp(nch - 1, 1).wait()
    fo_cp(nch - 2, 0).wait()
    fo_cp(nch - 1, 1).wait()


@functools.cache
def _build_gather(goff, nch):
    mesh = plsc.VectorSubcoreMesh(
        core_axis_name="c", subcore_axis_name="s",
        num_cores=NC, num_subcores=NS,
    )
    bf_s = nch * CHUNK * NW
    return pl.kernel(
        functools.partial(_gather_body, goff, nch),
        out_type=(
            jax.ShapeDtypeStruct((bf_s * K // 128, 128), jnp.float32),
            jax.ShapeDtypeStruct((bf_s,), jnp.float32),
        ),
        mesh=mesh,
        compiler_params=pltpu.CompilerParams(use_tc_tiling_on_sc=True, needs_layout_passes=False),
        scratch_types=(
            pltpu.VMEM((nch * CHUNK,), jnp.int32),
            pltpu.VMEM((2, CHUNK), jnp.int32),
            pltpu.VMEM((2, CHUNK, 128), jnp.float32),
            pltpu.VMEM((2, 16, 128), jnp.float32),
            pltpu.VMEM((2, CHUNK), jnp.float32),
            (pltpu.SemaphoreType.DMA, pltpu.SemaphoreType.DMA),
            (pltpu.SemaphoreType.DMA, pltpu.SemaphoreType.DMA),
            (pltpu.SemaphoreType.DMA, pltpu.SemaphoreType.DMA),
            (pltpu.SemaphoreType.DMA, pltpu.SemaphoreType.DMA),
        ),
    )


BLK = 1024


def _mlp_body(e_ref, fc_ref, s_ref, w1_ref, b1_ref, w2_ref, b2_ref,
              w3_ref, b3_ref, wlin_ref, blin_ref, out_ref):
    e = e_ref[...]
    s = s_ref[...]
    sum_f = lax.dot_general(e, s, (((1,), (0,)), ((), ())),
                            preferred_element_type=jnp.float32)
    ssq = lax.dot_general(e * e, s, (((1,), (0,)), ((), ())),
                          preferred_element_type=jnp.float32)
    fm = 0.5 * jnp.sum(sum_f * sum_f - ssq, axis=1, keepdims=True)

    lin = jnp.sum(fc_ref[...], axis=1, keepdims=True)
    lin = lin * wlin_ref[0, 0] + blin_ref[0, 0]

    h = lax.dot_general(e, w1_ref[...], (((1,), (0,)), ((), ())),
                        preferred_element_type=jnp.float32)
    h = jnp.maximum(h + b1_ref[...], 0.0)
    h = lax.dot_general(h, w2_ref[...], (((1,), (0,)), ((), ())),
                        preferred_element_type=jnp.float32)
    h = jnp.maximum(h + b2_ref[...], 0.0)
    mlp = lax.dot_general(h, w3_ref[...], (((1,), (0,)), ((), ())),
                          preferred_element_type=jnp.float32)
    mlp = mlp + b3_ref[0, 0]

    z = lin + fm + mlp
    out_ref[...] = 1.0 / (1.0 + jnp.exp(-z))


def _tc_mlp(e, fcm, s, w1, b1, w2, b2, w3, b3, wlin, blin):
    nb = e.shape[0]
    grid = (nb // BLK,)
    fixed = lambda i: (0, 0)
    return pl.pallas_call(
        _mlp_body,
        grid=grid,
        in_specs=[
            pl.BlockSpec((BLK, EMBED_OUT), lambda i: (i, 0)),
            pl.BlockSpec((BLK, F), lambda i: (i, 0)),
            pl.BlockSpec((EMBED_OUT, K), fixed),
            pl.BlockSpec((EMBED_OUT, 128), fixed),
            pl.BlockSpec((1, 128), fixed),
            pl.BlockSpec((128, 64), fixed),
            pl.BlockSpec((1, 64), fixed),
            pl.BlockSpec((64, 1), fixed),
            pl.BlockSpec((1, 1), fixed),
            pl.BlockSpec((1, 1), fixed),
            pl.BlockSpec((1, 1), fixed),
        ],
        out_specs=pl.BlockSpec((BLK, 1), lambda i: (i, 0)),
        out_shape=jax.ShapeDtypeStruct((nb, 1), jnp.float32),
    )(e, fcm, s, w1, b1, w2, b2, w3, b3, wlin, blin)


def kernel(x, embedding, fc_table, w_lin, b_lin, W1, b1, W2, b2, W3, b3):
    xf = x.reshape(-1).astype(jnp.int32)
    tail_piece = jnp.pad(embedding[NPAIR_FULL * 128:],
                         ((0, 128 - TAIL), (0, 0))).reshape(16, 128)
    tab = _build_prepass()(embedding.T, tail_piece)
    fct = fc_table.reshape(-1)
    s = (jnp.arange(EMBED_OUT)[:, None] % K == jnp.arange(K)[None, :]
         ).astype(jnp.float32)
    outs = []
    for goff, nch in SLICES:
        rows, fcv = _build_gather(goff, nch)(xf, tab, fct)
        nb = nch * CHUNK * NW // F
        e = rows.reshape(nb, EMBED_OUT)
        fcm = fcv.reshape(nb, F)
        outs.append(_tc_mlp(e, fcm, s, W1, b1.reshape(1, -1),
                            W2, b2.reshape(1, -1), W3, b3.reshape(1, 1),
                            w_lin, b_lin.reshape(1, 1)))
    return jnp.concatenate(outs, axis=0)


# speedup vs baseline: 1.0721x; 1.0028x over previous
import functools

import jax
import jax.numpy as jnp
from jax import lax
from jax.experimental import pallas as pl
from jax.experimental.pallas import tpu as pltpu
from jax.experimental.pallas import tpu_sc as plsc

B = 16384
F = 26
K = 16
EMBED_OUT = F * K
BF = B * F
NV = 1000012

NC = 2
NS = 16
NW = NC * NS
PER_W = BF // NW
CHUNK = 128
NCHUNK = PER_W // CHUNK

NPAIR = 7813
NPAIR_FULL = 7812
TAIL = NV - NPAIR_FULL * 128
NGROUP = NPAIR * 16
OUT_ROWS = BF * K // 128


_BASEC = 244
_EXTRA0 = _BASEC * NW
CP = 4
_NCH_PRE = _BASEC // CP


def _permute_tile(src_ref, dst_ref, src_col0, dst_row0):
    lane = lax.iota(jnp.int32, 16)
    rbase = jax.lax.shift_right_logical(lane, 3) + dst_row0
    cbase = jax.lax.bitwise_and(lane, 7) * 16
    for q in range(8):
        r = rbase + 2 * q
        for k in range(16):
            v = src_ref[k, pl.ds(src_col0 + q * 16, 16)]
            plsc.store_scatter(dst_ref, [r, cbase + k], v)


def _prepass_body(embt_hbm, tailp_hbm, tab_hbm, tile_v, piece_v,
                  sem_in, sem_out):
    wid = lax.axis_index("s") * NC + lax.axis_index("c")
    pstart = _BASEC * wid

    def in_cp(c, b):
        p0 = pstart + c * CP
        return pltpu.make_async_copy(
            embt_hbm.at[:, pl.ds(pl.multiple_of(p0 * 128, 128), CP * 128)], tile_v.at[b], sem_in[b])

    def out_cp(c, b):
        p0 = pstart + c * CP
        return pltpu.make_async_copy(
            piece_v.at[b],
            tab_hbm.at[pl.ds(pl.multiple_of(p0 * 16, 16), CP * 16)],
            sem_out[b])

    in_cp(0, 0).start()

    @pl.loop(0, _NCH_PRE // 2 + 1)
    def _j(j):
        for b in range(2):
            c = 2 * j + b

            @pl.when(c < _NCH_PRE)
            def _do():
                in_cp(c, b).wait()

                @pl.when(c + 1 < _NCH_PRE)
                def _pf():
                    in_cp(c + 1, 1 - b).start()

                @pl.when(c > 1)
                def _wo():
                    out_cp(c - 2, b).wait()

                for pp in range(CP):
                    _permute_tile(tile_v.at[b], piece_v.at[b], pp * 128, pp * 16)
                out_cp(c, b).start()

    out_cp(_NCH_PRE - 2, _NCH_PRE % 2).wait()
    out_cp(_NCH_PRE - 1, (_NCH_PRE - 1) % 2).wait()

    nrem = _BASEC - _NCH_PRE * CP
    if nrem:
        p0 = pstart + _NCH_PRE * CP
        pltpu.async_copy(
            embt_hbm.at[:, pl.ds(pl.multiple_of(p0 * 128, 128), nrem * 128)],
            tile_v.at[0, :, pl.ds(0, nrem * 128)], sem_in[0]).wait()
        for pp in range(nrem):
            _permute_tile(tile_v.at[0], piece_v.at[0], pp * 128, pp * 16)
        pltpu.async_copy(
            piece_v.at[0, pl.ds(0, nrem * 16)],
            tab_hbm.at[pl.ds(pl.multiple_of(p0 * 16, 16), nrem * 16)],
            sem_out[0]).wait()

    @pl.when(wid < 4)
    def _extra():
        p = _EXTRA0 + wid
        pltpu.async_copy(
            embt_hbm.at[:, pl.ds(pl.multiple_of(p * 128, 128), 128)],
            tile_v.at[0, :, pl.ds(0, 128)], sem_in[0]).wait()
        _permute_tile(tile_v.at[0], piece_v.at[0], 0, 0)
        pltpu.async_copy(
            piece_v.at[0, pl.ds(0, 16)],
            tab_hbm.at[pl.ds(pl.multiple_of(p * 16, 16), 16)],
            sem_out[0]).wait()

    @pl.when(wid == 4)
    def _tail():
        pltpu.async_copy(tailp_hbm, piece_v.at[0, pl.ds(0, 16)],
                         sem_in[0]).wait()
        pltpu.async_copy(
            piece_v.at[0, pl.ds(0, 16)],
            tab_hbm.at[pl.ds(NPAIR_FULL * 16, 16)], sem_out[0]).wait()


@functools.cache
def _build_prepass():
    mesh = plsc.VectorSubcoreMesh(
        core_axis_name="c", subcore_axis_name="s",
        num_cores=NC, num_subcores=NS,
    )
    return pl.kernel(
        _prepass_body,
        out_type=jax.ShapeDtypeStruct((NGROUP, 128), jnp.float32),
        mesh=mesh,
        compiler_params=pltpu.CompilerParams(use_tc_tiling_on_sc=True, needs_layout_passes=False),
        scratch_types=(
            pltpu.VMEM((2, 16, CP * 128), jnp.float32),
            pltpu.VMEM((2, CP * 16, 128), jnp.float32),
            (pltpu.SemaphoreType.DMA, pltpu.SemaphoreType.DMA),
            (pltpu.SemaphoreType.DMA, pltpu.SemaphoreType.DMA),
        ),
    )


SLICES = ((0, 52), (52 * CHUNK * NW, 52))


def _gather_body(goff, nch, idx_hbm, tab_hbm, fct_hbm, rows_hbm, fcv_hbm,
                 idx_v, g_v, grp_v, out_v, fcv_v,
                 sem_g, sem_f, sem_o, sem_fo):
    wid = lax.axis_index("s") * NC + lax.axis_index("c")
    per_w = nch * CHUNK
    base = wid * per_w
    gbase = goff + base
    pltpu.sync_copy(idx_hbm.at[pl.ds(gbase, per_w)], idx_v)

    def build_g(c, b):
        off = c * CHUNK
        for q in range(CHUNK // 16):
            iv = idx_v[pl.ds(off + q * 16, 16)]
            g_v[b, pl.ds(q * 16, 16)] = jax.lax.shift_right_logical(iv, 3)

    def g_cp(c, b):
        return pltpu.make_async_copy(tab_hbm.at[g_v.at[b]], grp_v.at[b],
                                     sem_g[b])

    def f_cp(c, b):
        off = c * CHUNK
        return pltpu.make_async_copy(
            fct_hbm.at[idx_v.at[pl.ds(off, CHUNK)]], fcv_v.at[b], sem_f[b])

    def o_cp(c, b):
        off = c * CHUNK
        return pltpu.make_async_copy(
            out_v.at[b],
            rows_hbm.at[pl.ds(pl.multiple_of((base + off) // 8, CHUNK // 8), CHUNK // 8)],
            sem_o[b])

    def fo_cp(c, b):
        off = c * CHUNK
        return pltpu.make_async_copy(
            fcv_v.at[b], fcv_hbm.at[pl.ds(base + off, CHUNK)], sem_fo[b])

    build_g(0, 0)
    g_cp(0, 0).start()
    f_cp(0, 0).start()

    @pl.loop(0, nch // 2)
    def _j(j):
        for b in range(2):
            c = 2 * j + b
            @pl.when(c + 1 < nch)
            def _pf():
                @pl.when(c > 0)
                def _wfo():
                    fo_cp(c - 1, 1 - b).wait()
                build_g(c + 1, 1 - b)
                g_cp(c + 1, 1 - b).start()
                f_cp(c + 1, 1 - b).start()

            g_cp(c, b).wait()
            f_cp(c, b).wait()

            @pl.when(c > 1)
            def _wo():
                o_cp(c - 2, b).wait()

            off = c * CHUNK
            lane = lax.iota(jnp.int32, 16)
            rbase = jax.lax.shift_right_logical(lane, 3)
            cbase = jax.lax.bitwise_and(lane, 7) * 16
            for q in range(CHUNK // 16):
                iv = idx_v[pl.ds(off + q * 16, 16)]
                sub = jax.lax.bitwise_and(iv, 7) * 16
                j16 = jnp.full((16,), q * 16, jnp.int32) + lane
                r = rbase + 2 * q
                for k in range(16):
                    v = plsc.load_gather(grp_v.at[b], [j16, sub + k])
                    plsc.store_scatter(out_v.at[b], [r, cbase + k], v)
            o_cp(c, b).start()
            fo_cp(c, b).start()

    o_cp(nch - 2, 0).wait()
    o_cp(nch - 1, 1).wait()
    fo_cp(nch - 2, 0).wait()
    fo_cp(nch - 1, 1).wait()


@functools.cache
def _build_gather(goff, nch):
    mesh = plsc.VectorSubcoreMesh(
        core_axis_name="c", subcore_axis_name="s",
        num_cores=NC, num_subcores=NS,
    )
    bf_s = nch * CHUNK * NW
    return pl.kernel(
        functools.partial(_gather_body, goff, nch),
        out_type=(
            jax.ShapeDtypeStruct((bf_s * K // 128, 128), jnp.float32),
            jax.ShapeDtypeStruct((bf_s,), jnp.float32),
        ),
        mesh=mesh,
        compiler_params=pltpu.CompilerParams(use_tc_tiling_on_sc=True, needs_layout_passes=False),
        scratch_types=(
            pltpu.VMEM((nch * CHUNK,), jnp.int32),
            pltpu.VMEM((2, CHUNK), jnp.int32),
            pltpu.VMEM((2, CHUNK, 128), jnp.float32),
            pltpu.VMEM((2, CHUNK // 8, 128), jnp.float32),
            pltpu.VMEM((2, CHUNK), jnp.float32),
            (pltpu.SemaphoreType.DMA, pltpu.SemaphoreType.DMA),
            (pltpu.SemaphoreType.DMA, pltpu.SemaphoreType.DMA),
            (pltpu.SemaphoreType.DMA, pltpu.SemaphoreType.DMA),
            (pltpu.SemaphoreType.DMA, pltpu.SemaphoreType.DMA),
        ),
    )


BLK = 1024


def _mlp_body(e_ref, fc_ref, s_ref, w1_ref, b1_ref, w2_ref, b2_ref,
              w3_ref, b3_ref, wlin_ref, blin_ref, out_ref):
    e = e_ref[...]
    s = s_ref[...]
    sum_f = lax.dot_general(e, s, (((1,), (0,)), ((), ())),
                            preferred_element_type=jnp.float32)
    ssq = lax.dot_general(e * e, s, (((1,), (0,)), ((), ())),
                          preferred_element_type=jnp.float32)
    fm = 0.5 * jnp.sum(sum_f * sum_f - ssq, axis=1, keepdims=True)

    lin = jnp.sum(fc_ref[...], axis=1, keepdims=True)
    lin = lin * wlin_ref[0, 0] + blin_ref[0, 0]

    h = lax.dot_general(e, w1_ref[...], (((1,), (0,)), ((), ())),
                        preferred_element_type=jnp.float32)
    h = jnp.maximum(h + b1_ref[...], 0.0)
    h = lax.dot_general(h, w2_ref[...], (((1,), (0,)), ((), ())),
                        preferred_element_type=jnp.float32)
    h = jnp.maximum(h + b2_ref[...], 0.0)
    mlp = lax.dot_general(h, w3_ref[...], (((1,), (0,)), ((), ())),
                          preferred_element_type=jnp.float32)
    mlp = mlp + b3_ref[0, 0]

    z = lin + fm + mlp
    out_ref[...] = 1.0 / (1.0 + jnp.exp(-z))


def _tc_mlp(e, fcm, s, w1, b1, w2, b2, w3, b3, wlin, blin):
    nb = e.shape[0]
    grid = (nb // BLK,)
    fixed = lambda i: (0, 0)
    return pl.pallas_call(
        _mlp_body,
        grid=grid,
        in_specs=[
            pl.BlockSpec((BLK, EMBED_OUT), lambda i: (i, 0)),
            pl.BlockSpec((BLK, F), lambda i: (i, 0)),
            pl.BlockSpec((EMBED_OUT, K), fixed),
            pl.BlockSpec((EMBED_OUT, 128), fixed),
            pl.BlockSpec((1, 128), fixed),
            pl.BlockSpec((128, 64), fixed),
            pl.BlockSpec((1, 64), fixed),
            pl.BlockSpec((64, 1), fixed),
            pl.BlockSpec((1, 1), fixed),
            pl.BlockSpec((1, 1), fixed),
            pl.BlockSpec((1, 1), fixed),
        ],
        out_specs=pl.BlockSpec((BLK, 1), lambda i: (i, 0)),
        out_shape=jax.ShapeDtypeStruct((nb, 1), jnp.float32),
    )(e, fcm, s, w1, b1, w2, b2, w3, b3, wlin, blin)


def kernel(x, embedding, fc_table, w_lin, b_lin, W1, b1, W2, b2, W3, b3):
    xf = x.reshape(-1).astype(jnp.int32)
    tail_piece = jnp.pad(embedding[NPAIR_FULL * 128:],
                         ((0, 128 - TAIL), (0, 0))).reshape(16, 128)
    tab = _build_prepass()(embedding.T, tail_piece)
    fct = fc_table.reshape(-1)
    s = (jnp.arange(EMBED_OUT)[:, None] % K == jnp.arange(K)[None, :]
         ).astype(jnp.float32)
    outs = []
    for goff, nch in SLICES:
        rows, fcv = _build_gather(goff, nch)(xf, tab, fct)
        nb = nch * CHUNK * NW // F
        e = rows.reshape(nb, EMBED_OUT)
        fcm = fcv.reshape(nb, F)
        outs.append(_tc_mlp(e, fcm, s, W1, b1.reshape(1, -1),
                            W2, b2.reshape(1, -1), W3, b3.reshape(1, 1),
                            w_lin, b_lin.reshape(1, 1)))
    return jnp.concatenate(outs, axis=0)
